# baseline (speedup 1.0000x reference)
import jax
import jax.numpy as jnp
from jax.experimental import pallas as pl
from jax.experimental.pallas import tpu as pltpu

_N = 2048
_T = 12
_H = 64


def _b16(x):
    return x.astype(jnp.bfloat16).astype(jnp.float32)


def _dot(a, b):
    return jnp.dot(a, b, preferred_element_type=jnp.float32)


def _body(xst_ref, adj_ref, w1xg_ref, w1hg_ref, b1g_ref,
          w1xc_ref, w1hc_ref, b1c_ref,
          wh23g_ref, wx2g_ref, wx3g_ref, b2g_ref, b3g_ref,
          wh23c_ref, wx23c_ref, b23c_ref,
          outb_ref, outu_ref,
          ax_ref, h1_ref, ah1_ref, h23_ref, u23_ref, c23h_ref, q_ref):
    adj = adj_ref[...]
    ax_ref[...] = _dot(adj, xst_ref[...])
    h1_ref[...] = jnp.zeros((_N, _H), jnp.float32)
    ah1_ref[...] = jnp.zeros((_N, _H), jnp.float32)
    h23_ref[...] = jnp.zeros((_N, 2), jnp.float32)
    u23_ref[...] = jnp.ones((_N, 2), jnp.float32)
    c23h_ref[...] = jnp.zeros((_N, 2), jnp.float32)
    q_ref[...] = jnp.zeros((_N, 2), jnp.float32)

    w1xg = w1xg_ref[...]
    w1hg = w1hg_ref[...]
    b1g = b1g_ref[...]
    w1xc = w1xc_ref[...]
    w1hc = w1hc_ref[...]
    b1c = b1c_ref[...]
    wh23g = wh23g_ref[...]
    wx2g = wx2g_ref[...]
    wx3g = wx3g_ref[...]
    b2g = b2g_ref[...]
    b3g = b3g_ref[...]
    wh23c = wh23c_ref[...]
    wx23c = wx23c_ref[...]
    b23c = b23c_ref[...]

    lane_t = jax.lax.broadcasted_iota(jnp.int32, (_N, _T), 1)
    row_t = jax.lax.broadcasted_iota(jnp.int32, (_T, 1), 0)
    wh23gc = jnp.concatenate([wh23g, wh23c], axis=1)

    def finish_prev(aq_prev, sel_prev):
        c23p = jnp.tanh(c23h_ref[...] + aq_prev * wx23c + b23c)
        u23p = u23_ref[...]
        h23 = u23p * h23_ref[...] + (1.0 - u23p) * c23p
        h23_ref[...] = h23
        outb_ref[...] = jnp.where(sel_prev, h23[:, 0:1], outb_ref[...])
        outu_ref[...] = jnp.where(sel_prev, h23[:, 1:2], outu_ref[...])
        return h23

    def step(t, _):
        h1 = h1_ref[...]
        onehot = (row_t == t).astype(jnp.float32)
        ax = _dot(ax_ref[...], onehot)
        g1 = jax.nn.sigmoid(ax * w1xg + _dot(ah1_ref[...], w1hg) + b1g)
        r1 = g1[:, :_H]
        u1 = g1[:, _H:]
        mA = jnp.concatenate([r1 * h1, q_ref[...]], axis=1)
        PA = _dot(adj, mA)
        arh = PA[:, :_H]
        aq_prev = _b16(PA[:, _H:_H + 2])
        h23 = finish_prev(aq_prev, lane_t == t - 1)
        c1 = jnp.tanh(ax * w1xc + _dot(arh, w1hc) + b1c)
        h1n = u1 * h1 + (1.0 - u1) * c1
        h1_ref[...] = h1n
        mB = jnp.concatenate([h1n, h23], axis=1)
        PB = _dot(adj, mB)
        ah1n = PB[:, :_H]
        ah23 = _b16(PB[:, _H:_H + 2])
        ah1_ref[...] = ah1n
        g23hc = _dot(ah1n, wh23gc)
        g2 = jax.nn.sigmoid(g23hc[:, 0:2] + ah23[:, 0:1] * wx2g + b2g)
        g3 = jax.nn.sigmoid(g23hc[:, 2:4] + ah23[:, 1:2] * wx3g + b3g)
        u23_ref[...] = jnp.concatenate([g2[:, 1:2], g3[:, 1:2]], axis=1)
        r23 = jnp.concatenate([g2[:, 0:1], g3[:, 0:1]], axis=1)
        q_ref[...] = r23 * h23
        c23h_ref[...] = g23hc[:, 4:6]
        return 0

    jax.lax.fori_loop(0, _T, step, 0)
    aq = _b16(_dot(adj, q_ref[...]))
    finish_prev(aq, lane_t == _T - 1)


def kernel(features, adj, W1_g, b1_g, W1_c, b1_c, W2_g, b2_g, W2_c, b2_c,
           W3_g, b3_g, W3_c, b3_c):
    xst = features[0].T
    bf = lambda a: a.astype(jnp.bfloat16).astype(jnp.float32)
    w1xg = bf(W1_g[0:1, :])
    w1hg = W1_g[1:, :]
    w1xc = bf(W1_c[0:1, :])
    w1hc = W1_c[1:, :]
    wh23g = jnp.concatenate([W2_g[:_H], W3_g[:_H]], axis=1)
    wx2g = bf(W2_g[_H:_H + 1, :])
    wx3g = bf(W3_g[_H:_H + 1, :])
    wh23c = jnp.concatenate([W2_c[:_H], W3_c[:_H]], axis=1)
    wx23c = bf(jnp.concatenate([W2_c[_H:], W3_c[_H:]], axis=1))
    b23c = jnp.concatenate([b2_c, b3_c])[None, :]

    outb, outu = pl.pallas_call(
        _body,
        out_shape=[jax.ShapeDtypeStruct((_N, _T), jnp.float32),
                   jax.ShapeDtypeStruct((_N, _T), jnp.float32)],
        scratch_shapes=[pltpu.VMEM((_N, _T), jnp.float32),
                        pltpu.VMEM((_N, _H), jnp.float32),
                        pltpu.VMEM((_N, _H), jnp.float32),
                        pltpu.VMEM((_N, 2), jnp.float32),
                        pltpu.VMEM((_N, 2), jnp.float32),
                        pltpu.VMEM((_N, 2), jnp.float32),
                        pltpu.VMEM((_N, 2), jnp.float32)],
        compiler_params=pltpu.CompilerParams(
            vmem_limit_bytes=100 * 1024 * 1024),
    )(xst, adj, w1xg, w1hg, b1_g[None, :], w1xc, w1hc, b1_c[None, :],
      wh23g, wx2g, wx3g, b2_g[None, :], b3_g[None, :],
      wh23c, wx23c, b23c)
    return outb.T[None], outu.T[None]


# speedup vs baseline: 1.0075x; 1.0075x over previous
import jax
import jax.numpy as jnp
from jax.experimental import pallas as pl
from jax.experimental.pallas import tpu as pltpu

_N = 2048
_T = 12
_H = 64


def _b16(x):
    return x.astype(jnp.bfloat16).astype(jnp.float32)


def _dot(a, b):
    return jnp.dot(a, b, preferred_element_type=jnp.float32)


def _body(xst_ref, adj_ref, w1xg_ref, w1hg_ref, b1g_ref,
          w1xc_ref, w1hc_ref, b1c_ref,
          wh23g_ref, wx2g_ref, wx3g_ref, b2g_ref, b3g_ref,
          wh23c_ref, wx23c_ref, b23c_ref,
          outb_ref, outu_ref,
          ax_ref, h1_ref, ah1_ref, h23_ref, u23_ref, c23h_ref, q_ref):
    adj = adj_ref[...]
    ax_ref[...] = _dot(adj, xst_ref[...])
    h1_ref[...] = jnp.zeros((_N, _H), jnp.float32)
    ah1_ref[...] = jnp.zeros((_N, _H), jnp.float32)
    h23_ref[...] = jnp.zeros((_N, 2), jnp.float32)
    u23_ref[...] = jnp.ones((_N, 2), jnp.float32)
    c23h_ref[...] = jnp.zeros((_N, 2), jnp.float32)
    q_ref[...] = jnp.zeros((_N, 2), jnp.float32)

    w1xg = w1xg_ref[...]
    w1hg = w1hg_ref[...]
    b1g = b1g_ref[...]
    w1xc = w1xc_ref[...]
    w1hc = w1hc_ref[...]
    b1c = b1c_ref[...]
    wh23g = wh23g_ref[...]
    wx2g = wx2g_ref[...]
    wx3g = wx3g_ref[...]
    b2g = b2g_ref[...]
    b3g = b3g_ref[...]
    wh23c = wh23c_ref[...]
    wx23c = wx23c_ref[...]
    b23c = b23c_ref[...]

    lane_t = jax.lax.broadcasted_iota(jnp.int32, (_N, _T), 1)
    row_t = jax.lax.broadcasted_iota(jnp.int32, (_T, 1), 0)
    wh23gc = jnp.concatenate([wh23g, wh23c], axis=1)

    def finish_prev(aq_prev, sel_prev):
        c23p = jnp.tanh(c23h_ref[...] + aq_prev * wx23c + b23c)
        u23p = u23_ref[...]
        h23 = u23p * h23_ref[...] + (1.0 - u23p) * c23p
        h23_ref[...] = h23
        outb_ref[...] = jnp.where(sel_prev, h23[:, 0:1], outb_ref[...])
        outu_ref[...] = jnp.where(sel_prev, h23[:, 1:2], outu_ref[...])
        return h23

    def step(t, _):
        h1 = h1_ref[...]
        sel = lane_t == t
        ax = _b16(jnp.sum(jnp.where(sel, ax_ref[...], 0.0), axis=1,
                          keepdims=True))
        g1 = jax.nn.sigmoid(ax * w1xg + _dot(ah1_ref[...], w1hg) + b1g)
        r1 = g1[:, :_H]
        u1 = g1[:, _H:]
        mA = jnp.concatenate([r1 * h1, q_ref[...]], axis=1)
        PA = _dot(adj, mA)
        arh = PA[:, :_H]
        aq_prev = _b16(PA[:, _H:_H + 2])
        h23 = finish_prev(aq_prev, lane_t == t - 1)
        c1 = jnp.tanh(ax * w1xc + _dot(arh, w1hc) + b1c)
        h1n = u1 * h1 + (1.0 - u1) * c1
        h1_ref[...] = h1n
        mB = jnp.concatenate([h1n, h23], axis=1)
        PB = _dot(adj, mB)
        ah1n = PB[:, :_H]
        ah23 = _b16(PB[:, _H:_H + 2])
        ah1_ref[...] = ah1n
        g23hc = _dot(ah1n, wh23gc)
        g2 = jax.nn.sigmoid(g23hc[:, 0:2] + ah23[:, 0:1] * wx2g + b2g)
        g3 = jax.nn.sigmoid(g23hc[:, 2:4] + ah23[:, 1:2] * wx3g + b3g)
        u23_ref[...] = jnp.concatenate([g2[:, 1:2], g3[:, 1:2]], axis=1)
        r23 = jnp.concatenate([g2[:, 0:1], g3[:, 0:1]], axis=1)
        q_ref[...] = r23 * h23
        c23h_ref[...] = g23hc[:, 4:6]
        return 0

    jax.lax.fori_loop(0, _T, step, 0)
    aq = _b16(_dot(adj, q_ref[...]))
    finish_prev(aq, lane_t == _T - 1)


def kernel(features, adj, W1_g, b1_g, W1_c, b1_c, W2_g, b2_g, W2_c, b2_c,
           W3_g, b3_g, W3_c, b3_c):
    xst = features[0].T
    bf = lambda a: a.astype(jnp.bfloat16).astype(jnp.float32)
    w1xg = bf(W1_g[0:1, :])
    w1hg = W1_g[1:, :]
    w1xc = bf(W1_c[0:1, :])
    w1hc = W1_c[1:, :]
    wh23g = jnp.concatenate([W2_g[:_H], W3_g[:_H]], axis=1)
    wx2g = bf(W2_g[_H:_H + 1, :])
    wx3g = bf(W3_g[_H:_H + 1, :])
    wh23c = jnp.concatenate([W2_c[:_H], W3_c[:_H]], axis=1)
    wx23c = bf(jnp.concatenate([W2_c[_H:], W3_c[_H:]], axis=1))
    b23c = jnp.concatenate([b2_c, b3_c])[None, :]

    outb, outu = pl.pallas_call(
        _body,
        out_shape=[jax.ShapeDtypeStruct((_N, _T), jnp.float32),
                   jax.ShapeDtypeStruct((_N, _T), jnp.float32)],
        scratch_shapes=[pltpu.VMEM((_N, _T), jnp.float32),
                        pltpu.VMEM((_N, _H), jnp.float32),
                        pltpu.VMEM((_N, _H), jnp.float32),
                        pltpu.VMEM((_N, 2), jnp.float32),
                        pltpu.VMEM((_N, 2), jnp.float32),
                        pltpu.VMEM((_N, 2), jnp.float32),
                        pltpu.VMEM((_N, 2), jnp.float32)],
        compiler_params=pltpu.CompilerParams(
            vmem_limit_bytes=100 * 1024 * 1024),
    )(xst, adj, w1xg, w1hg, b1_g[None, :], w1xc, w1hc, b1_c[None, :],
      wh23g, wx2g, wx3g, b2_g[None, :], b3_g[None, :],
      wh23c, wx23c, b23c)
    return outb.T[None], outu.T[None]


# speedup vs baseline: 1.0788x; 1.0708x over previous
import jax
import jax.numpy as jnp
from jax.experimental import pallas as pl
from jax.experimental.pallas import tpu as pltpu

_N = 2048
_T = 12
_H = 64


def _b16(x):
    return x.astype(jnp.bfloat16).astype(jnp.float32)


def _dot(a, b):
    return jnp.dot(a, b, preferred_element_type=jnp.float32)


def _body(xst_ref, adj_ref, w1xg_ref, w1hg_ref, b1g_ref,
          w1xc_ref, w1hc_ref, b1c_ref,
          wh23g_ref, wx2g_ref, wx3g_ref, b2g_ref, b3g_ref,
          wh23c_ref, wx23c_ref, b23c_ref,
          outb_ref, outu_ref,
          ax_ref, h1_ref, ah1_ref, h23_ref, u23_ref, c23h_ref, q_ref):
    adj = adj_ref[...]
    ax_ref[...] = _dot(adj, xst_ref[...])
    h1_ref[...] = jnp.zeros((_N, _H), jnp.float32)
    ah1_ref[...] = jnp.zeros((_N, _H), jnp.float32)
    h23_ref[...] = jnp.zeros((_N, 2), jnp.float32)
    u23_ref[...] = jnp.ones((_N, 2), jnp.float32)
    c23h_ref[...] = jnp.zeros((_N, 2), jnp.float32)
    q_ref[...] = jnp.zeros((_N, 2), jnp.float32)

    w1xg = w1xg_ref[...]
    w1hg = w1hg_ref[...]
    b1g = b1g_ref[...]
    w1xc = w1xc_ref[...]
    w1hc = w1hc_ref[...]
    b1c = b1c_ref[...]
    wh23g = wh23g_ref[...]
    wx2g = wx2g_ref[...]
    wx3g = wx3g_ref[...]
    b2g = b2g_ref[...]
    b3g = b3g_ref[...]
    wh23c = wh23c_ref[...]
    wx23c = wx23c_ref[...]
    b23c = b23c_ref[...]

    lane_t = jax.lax.broadcasted_iota(jnp.int32, (_N, _T), 1)

    def finish_prev(aq_prev, sel_prev):
        c23p = jnp.tanh(c23h_ref[...] + aq_prev * wx23c + b23c)
        u23p = u23_ref[...]
        h23 = u23p * h23_ref[...] + (1.0 - u23p) * c23p
        h23_ref[...] = h23
        outb_ref[...] = jnp.where(sel_prev, h23[:, 0:1], outb_ref[...])
        outu_ref[...] = jnp.where(sel_prev, h23[:, 1:2], outu_ref[...])
        return h23

    def step(t, _):
        h1 = h1_ref[...]
        sel = lane_t == t
        ax = _b16(jnp.sum(jnp.where(sel, ax_ref[...], 0.0), axis=1,
                          keepdims=True))
        g1 = jax.nn.sigmoid(ax * w1xg + _dot(ah1_ref[...], w1hg) + b1g)
        r1 = g1[:, :_H]
        u1 = g1[:, _H:]
        mA = jnp.concatenate([r1 * h1, q_ref[...]], axis=1)
        PA = _dot(adj, mA)
        arh = PA[:, :_H]
        aq_prev = _b16(PA[:, _H:_H + 2])
        h23 = finish_prev(aq_prev, lane_t == t - 1)
        c1 = jnp.tanh(ax * w1xc + _dot(arh, w1hc) + b1c)
        h1n = u1 * h1 + (1.0 - u1) * c1
        h1_ref[...] = h1n
        mB = jnp.concatenate([h1n, h23], axis=1)
        PB = _dot(adj, mB)
        ah1n = PB[:, :_H]
        ah23 = _b16(PB[:, _H:_H + 2])
        ah1_ref[...] = ah1n
        g23h = _dot(ah1n, wh23g)
        g2 = jax.nn.sigmoid(g23h[:, 0:2] + ah23[:, 0:1] * wx2g + b2g)
        g3 = jax.nn.sigmoid(g23h[:, 2:4] + ah23[:, 1:2] * wx3g + b3g)
        u23_ref[...] = jnp.concatenate([g2[:, 1:2], g3[:, 1:2]], axis=1)
        r23 = jnp.concatenate([g2[:, 0:1], g3[:, 0:1]], axis=1)
        q_ref[...] = r23 * h23
        c23h_ref[...] = _dot(ah1n, wh23c)
        return 0

    def step2(i, _):
        step(2 * i, 0)
        step(2 * i + 1, 0)
        return 0

    jax.lax.fori_loop(0, _T // 2, step2, 0)
    aq = _b16(_dot(adj, q_ref[...]))
    finish_prev(aq, lane_t == _T - 1)


def kernel(features, adj, W1_g, b1_g, W1_c, b1_c, W2_g, b2_g, W2_c, b2_c,
           W3_g, b3_g, W3_c, b3_c):
    xst = features[0].T
    bf = lambda a: a.astype(jnp.bfloat16).astype(jnp.float32)
    w1xg = bf(W1_g[0:1, :])
    w1hg = W1_g[1:, :]
    w1xc = bf(W1_c[0:1, :])
    w1hc = W1_c[1:, :]
    wh23g = jnp.concatenate([W2_g[:_H], W3_g[:_H]], axis=1)
    wx2g = bf(W2_g[_H:_H + 1, :])
    wx3g = bf(W3_g[_H:_H + 1, :])
    wh23c = jnp.concatenate([W2_c[:_H], W3_c[:_H]], axis=1)
    wx23c = bf(jnp.concatenate([W2_c[_H:], W3_c[_H:]], axis=1))
    b23c = jnp.concatenate([b2_c, b3_c])[None, :]

    outb, outu = pl.pallas_call(
        _body,
        out_shape=[jax.ShapeDtypeStruct((_N, _T), jnp.float32),
                   jax.ShapeDtypeStruct((_N, _T), jnp.float32)],
        scratch_shapes=[pltpu.VMEM((_N, _T), jnp.float32),
                        pltpu.VMEM((_N, _H), jnp.float32),
                        pltpu.VMEM((_N, _H), jnp.float32),
                        pltpu.VMEM((_N, 2), jnp.float32),
                        pltpu.VMEM((_N, 2), jnp.float32),
                        pltpu.VMEM((_N, 2), jnp.float32),
                        pltpu.VMEM((_N, 2), jnp.float32)],
        compiler_params=pltpu.CompilerParams(
            vmem_limit_bytes=100 * 1024 * 1024),
    )(xst, adj, w1xg, w1hg, b1_g[None, :], w1xc, w1hc, b1_c[None, :],
      wh23g, wx2g, wx3g, b2_g[None, :], b3_g[None, :],
      wh23c, wx23c, b23c)
    return outb.T[None], outu.T[None]


# speedup vs baseline: 1.0826x; 1.0035x over previous
import jax
import jax.numpy as jnp
from jax.experimental import pallas as pl
from jax.experimental.pallas import tpu as pltpu

_N = 2048
_T = 12
_H = 64


def _b16(x):
    return x.astype(jnp.bfloat16).astype(jnp.float32)


def _dot(a, b):
    return jnp.dot(a, b, preferred_element_type=jnp.float32)


def _body(xst_ref, adj_ref, w1xg_ref, w1hg_ref, b1g_ref,
          w1xc_ref, w1hc_ref, b1c_ref,
          wh23g_ref, wx2g_ref, wx3g_ref, b2g_ref, b3g_ref,
          wh23c_ref, wx23c_ref, b23c_ref,
          outb_ref, outu_ref,
          ax_ref, h1_ref, ah1_ref, h23_ref, u23_ref, c23h_ref, q_ref):
    adj = adj_ref[...]
    ax_ref[...] = _dot(adj, xst_ref[...])
    h1_ref[...] = jnp.zeros((_N, _H), jnp.float32)
    ah1_ref[...] = jnp.zeros((_N, _H), jnp.float32)
    h23_ref[...] = jnp.zeros((_N, 2), jnp.float32)
    u23_ref[...] = jnp.ones((_N, 2), jnp.float32)
    c23h_ref[...] = jnp.zeros((_N, 2), jnp.float32)
    q_ref[...] = jnp.zeros((_N, 2), jnp.float32)

    w1xg = w1xg_ref[...]
    w1hg = w1hg_ref[...]
    b1g = b1g_ref[...]
    w1xc = w1xc_ref[...]
    w1hc = w1hc_ref[...]
    b1c = b1c_ref[...]
    wh23g = wh23g_ref[...]
    wx2g = wx2g_ref[...]
    wx3g = wx3g_ref[...]
    b2g = b2g_ref[...]
    b3g = b3g_ref[...]
    wh23c = wh23c_ref[...]
    wx23c = wx23c_ref[...]
    b23c = b23c_ref[...]

    lane_t = jax.lax.broadcasted_iota(jnp.int32, (_N, _T), 1)

    def finish_prev(aq_prev, sel_prev):
        c23p = jnp.tanh(c23h_ref[...] + aq_prev * wx23c + b23c)
        u23p = u23_ref[...]
        h23 = u23p * h23_ref[...] + (1.0 - u23p) * c23p
        h23_ref[...] = h23
        outb_ref[...] = jnp.where(sel_prev, h23[:, 0:1], outb_ref[...])
        outu_ref[...] = jnp.where(sel_prev, h23[:, 1:2], outu_ref[...])
        return h23

    def step(t, _):
        h1 = h1_ref[...]
        sel = lane_t == t
        ax = _b16(jnp.sum(jnp.where(sel, ax_ref[...], 0.0), axis=1,
                          keepdims=True))
        g1 = jax.nn.sigmoid(ax * w1xg + _dot(ah1_ref[...], w1hg) + b1g)
        r1 = g1[:, :_H]
        u1 = g1[:, _H:]
        mA = jnp.concatenate([r1 * h1, q_ref[...]], axis=1)
        PA = _dot(adj, mA)
        arh = PA[:, :_H]
        aq_prev = _b16(PA[:, _H:_H + 2])
        h23 = finish_prev(aq_prev, lane_t == t - 1)
        c1 = jnp.tanh(ax * w1xc + _dot(arh, w1hc) + b1c)
        h1n = u1 * h1 + (1.0 - u1) * c1
        h1_ref[...] = h1n
        mB = jnp.concatenate([h1n, h23], axis=1)
        PB = _dot(adj, mB)
        ah1n = PB[:, :_H]
        ah23 = _b16(PB[:, _H:_H + 2])
        ah1_ref[...] = ah1n
        g23h = _dot(ah1n, wh23g)
        g2 = jax.nn.sigmoid(g23h[:, 0:2] + ah23[:, 0:1] * wx2g + b2g)
        g3 = jax.nn.sigmoid(g23h[:, 2:4] + ah23[:, 1:2] * wx3g + b3g)
        u23_ref[...] = jnp.concatenate([g2[:, 1:2], g3[:, 1:2]], axis=1)
        r23 = jnp.concatenate([g2[:, 0:1], g3[:, 0:1]], axis=1)
        q_ref[...] = r23 * h23
        c23h_ref[...] = _dot(ah1n, wh23c)
        return 0

    def step2(i, _):
        step(3 * i, 0)
        step(3 * i + 1, 0)
        step(3 * i + 2, 0)
        return 0

    jax.lax.fori_loop(0, _T // 3, step2, 0)
    aq = _b16(_dot(adj, q_ref[...]))
    finish_prev(aq, lane_t == _T - 1)


def kernel(features, adj, W1_g, b1_g, W1_c, b1_c, W2_g, b2_g, W2_c, b2_c,
           W3_g, b3_g, W3_c, b3_c):
    xst = features[0].T
    bf = lambda a: a.astype(jnp.bfloat16).astype(jnp.float32)
    w1xg = bf(W1_g[0:1, :])
    w1hg = W1_g[1:, :]
    w1xc = bf(W1_c[0:1, :])
    w1hc = W1_c[1:, :]
    wh23g = jnp.concatenate([W2_g[:_H], W3_g[:_H]], axis=1)
    wx2g = bf(W2_g[_H:_H + 1, :])
    wx3g = bf(W3_g[_H:_H + 1, :])
    wh23c = jnp.concatenate([W2_c[:_H], W3_c[:_H]], axis=1)
    wx23c = bf(jnp.concatenate([W2_c[_H:], W3_c[_H:]], axis=1))
    b23c = jnp.concatenate([b2_c, b3_c])[None, :]

    outb, outu = pl.pallas_call(
        _body,
        out_shape=[jax.ShapeDtypeStruct((_N, _T), jnp.float32),
                   jax.ShapeDtypeStruct((_N, _T), jnp.float32)],
        scratch_shapes=[pltpu.VMEM((_N, _T), jnp.float32),
                        pltpu.VMEM((_N, _H), jnp.float32),
                        pltpu.VMEM((_N, _H), jnp.float32),
                        pltpu.VMEM((_N, 2), jnp.float32),
                        pltpu.VMEM((_N, 2), jnp.float32),
                        pltpu.VMEM((_N, 2), jnp.float32),
                        pltpu.VMEM((_N, 2), jnp.float32)],
        compiler_params=pltpu.CompilerParams(
            vmem_limit_bytes=100 * 1024 * 1024),
    )(xst, adj, w1xg, w1hg, b1_g[None, :], w1xc, w1hc, b1_c[None, :],
      wh23g, wx2g, wx3g, b2_g[None, :], b3_g[None, :],
      wh23c, wx23c, b23c)
    return outb.T[None], outu.T[None]
